# baseline (speedup 1.0000x reference)
import math
import functools

import jax
import jax.numpy as jnp
import numpy as np
from jax import lax
from jax.experimental import pallas as pl
from jax.experimental.pallas import tpu as pltpu
from jax.experimental.pallas import tpu_sc as plsc

_ELEV = np.array([-30.67, -29.33, -28.0, -26.66, -25.33, -24.0, -22.67, -21.33,
                  -20.0, -18.67, -17.33, -16.0, -14.67, -13.33, -12.0, -10.67,
                  -9.33, -8.0, -6.66, -5.33, -4.0, -2.67, -1.33, 0.0, 1.33,
                  2.67, 4.0, 5.33, 6.67, 8.0, 9.33, 10.67], dtype=np.float32)
_B, _HRV, _WRV, _CRV = 1, 32, 512, 64
_CBEV, _HBEV, _WBEV = 256, 256, 256
_D, _K, _COUT = 128, 5, 128
_RMAX, _BIN, _NBINS = 55.0, 0.5, 110
_XMIN, _XMAX, _YMIN, _YMAX = -55.0, 55.0, -55.0, 55.0
_NHEADS, _NPOINTS = 8, 6
_DH = _D // _NHEADS
_NQ = _K * _HRV * _WRV

_az_line = np.linspace(-math.pi, math.pi, _WRV + 1).astype(np.float32)[:-1]
_az = np.broadcast_to(_az_line.reshape(1, 1, 1, _WRV), (1, 1, _HRV, _WRV)) \
    + 0.5 * (_az_line[1] - _az_line[0])
_elev = np.deg2rad(np.array(np.split(_ELEV[::-1].copy(), _HRV)).mean(axis=1)
                   .repeat(_WRV).reshape(_HRV, _WRV)).astype(np.float32)
_U_X = (np.cos(_az) * np.cos(_elev)).astype(np.float32)
_U_Y = (np.sin(_az) * np.cos(_elev)).astype(np.float32)
_U_Z = np.sin(_elev)[None, None].astype(np.float32)
_U_VEC = np.concatenate([_U_X, _U_Y, _U_Z], axis=1)

_NC, _NS = 2, 16
_NW = _NC * _NS
_T = _NQ * _NHEADS * _NPOINTS
_TW = _T // _NW
_STEP_IT = 96
_SS_STEPS = 16
_SS_IT = _STEP_IT * _SS_STEPS
_N_SS = _TW // _SS_IT
_GROUPS = _T // _NPOINTS
_GW = _GROUPS // _NW
_SS_G = _SS_IT // _NPOINTS


def _msda_sample_sc(table, px, py, aw):
    mesh = plsc.VectorSubcoreMesh(core_axis_name="c", subcore_axis_name="s")

    @functools.partial(
        pl.kernel, mesh=mesh,
        out_type=jax.ShapeDtypeStruct((_GROUPS, _DH), jnp.float32),
        compiler_params=pltpu.CompilerParams(needs_layout_passes=False,
                                             use_tc_tiling_on_sc=False),
        scratch_types=[
            pltpu.VMEM((_SS_IT,), jnp.float32),
            pltpu.VMEM((_SS_IT,), jnp.float32),
            pltpu.VMEM((_SS_IT,), jnp.float32),
            pltpu.VMEM((3, 128), jnp.int32), pltpu.VMEM((3, 128), jnp.int32),
            pltpu.VMEM((4 * _STEP_IT,), jnp.float32),
            pltpu.VMEM((4 * _STEP_IT,), jnp.float32),
            pltpu.VMEM((4 * _STEP_IT, _DH), jnp.float32),
            pltpu.VMEM((4 * _STEP_IT, _DH), jnp.float32),
            pltpu.VMEM((_SS_G, _DH), jnp.float32),
            pltpu.SemaphoreType.DMA, pltpu.SemaphoreType.DMA,
        ],
    )
    def k(table_h, px_h, py_h, aw_h, out_h,
          pxb, pyb, awb, idxb0, idxb1,
          sb0, sb1, rb0, rb1, outb, sem0, sem1):
        idxbs = [idxb0, idxb1]
        sbs = [sb0, sb1]
        rbs = [rb0, rb1]
        sems = [sem0, sem1]
        wid = lax.axis_index("s") * _NC + lax.axis_index("c")
        t0w = wid * _TW
        g0w = wid * _GW
        iota = lax.iota(jnp.int32, 16)
        dcols = [jnp.bitwise_and(iota + k, 15) for k in range(_DH)]

        def ss_body(ss, carry):
            t0 = t0w + ss * _SS_IT
            pltpu.sync_copy(px_h.at[pl.ds(t0, _SS_IT)], pxb)
            pltpu.sync_copy(py_h.at[pl.ds(t0, _SS_IT)], pyb)
            pltpu.sync_copy(aw_h.at[pl.ds(t0, _SS_IT)], awb)

            def phase1(st, idxb, sb):
                for i in range(_STEP_IT // 16):
                    o = st * _STEP_IT + i * 16
                    ivec = o + iota
                    pxv = plsc.load_gather(pxb, [ivec])
                    pyv = plsc.load_gather(pyb, [ivec])
                    awv = plsc.load_gather(awb, [ivec])
                    t_v = t0 + ivec
                    h_v = jnp.bitwise_and(t_v // _NPOINTS, _NHEADS - 1)
                    hbv = jnp.left_shift(h_v, 16)
                    xt = pxv.astype(jnp.int32)
                    xtf = xt.astype(jnp.float32)
                    x0i = xt - jnp.where(pxv < xtf, 1, 0)
                    wx1 = pxv - x0i.astype(jnp.float32)
                    wx0 = 1.0 - wx1
                    yt = pyv.astype(jnp.int32)
                    ytf = yt.astype(jnp.float32)
                    y0i = yt - jnp.where(pyv < ytf, 1, 0)
                    wy1 = pyv - y0i.astype(jnp.float32)
                    wy0 = 1.0 - wy1
                    for c, (dx, dy) in enumerate(((0, 0), (1, 0), (0, 1), (1, 1))):
                        xi = x0i + dx
                        yi = y0i + dy
                        wx = wx1 if dx else wx0
                        wy = wy1 if dy else wy0
                        valid = ((xi >= 0) & (xi <= _WBEV - 1)
                                 & (yi >= 0) & (yi <= _HBEV - 1))
                        xc = jnp.clip(xi, 0, _WBEV - 1)
                        yc = jnp.clip(yi, 0, _HBEV - 1)
                        idx_c = hbv + yc * _WBEV + xc
                        s_c = jnp.where(valid, wx * wy * awv, 0.0)
                        flat = i * 64 + c * 16
                        idxb[flat // 128, pl.ds(flat % 128, 16)] = idx_c
                        sb[pl.ds(flat, 16)] = s_c

            def fire(idxb, rowsb, sem):
                return [pltpu.async_copy(table_h.at[idxb.at[c]],
                                         rowsb.at[pl.ds(c * 128, 128)], sem)
                        for c in range(3)]

            rbases = []
            for j6 in range(_NPOINTS):
                item_v = iota * _NPOINTS + j6
                i_v = jnp.right_shift(item_v, 4)
                lane_v = jnp.bitwise_and(item_v, 15)
                rbases.append(i_v * 64 + lane_v)

            def phase3(st, sb, rowsb):
                accs = [jnp.zeros((16,), jnp.float32) for _ in range(_DH)]
                for j in range(24):
                    j6, c = j // 4, j % 4
                    r_v = rbases[j6] + c * 16
                    s_j = plsc.load_gather(sb, [r_v])
                    for k in range(_DH):
                        g_v = plsc.load_gather(rowsb, [r_v, dcols[k]])
                        accs[k] = accs[k] + s_j * g_v
                grow = st * 16 + iota
                for k in range(_DH):
                    plsc.store_scatter(outb, [grow, dcols[k]], accs[k])

            def pair_body(qt, c2):
                handles = []
                for u in range(2):
                    st = qt * 2 + u
                    phase1(st, idxbs[u], sbs[u])
                    handles.append(fire(idxbs[u], rbs[u], sems[u]))
                for u in range(2):
                    for cp in handles[u]:
                        cp.wait()
                    phase3(qt * 2 + u, sbs[u], rbs[u])
                return c2

            lax.fori_loop(0, _SS_STEPS // 2, pair_body, 0)
            pltpu.sync_copy(outb, out_h.at[pl.ds(g0w + ss * _SS_G, _SS_G)])
            return carry

        lax.fori_loop(0, _N_SS, ss_body, 0)

    return k(table, px, py, aw)


def _conv1x1(x, w, b=None):
    y = jnp.einsum('bchw,oc->bohw', x, w)
    if b is not None:
        y = y + b[None, :, None, None]
    return y


def _gn(x, gamma, beta, groups, eps=1e-5):
    n, c, h, w = x.shape
    xg = x.reshape(n, groups, c // groups, h, w)
    m = xg.mean(axis=(2, 3, 4), keepdims=True)
    v = ((xg - m) ** 2).mean(axis=(2, 3, 4), keepdims=True)
    xg = (xg - m) / jnp.sqrt(v + eps)
    return xg.reshape(n, c, h, w) * gamma[None, :, None, None] + beta[None, :, None, None]


def _gelu(x):
    return jax.nn.gelu(x, approximate=False)


def _conv3x3_circ(x, w):
    xp = jnp.pad(x, ((0, 0), (0, 0), (1, 1), (1, 1)), mode='wrap')
    return jax.lax.conv_general_dilated(xp, w, (1, 1), 'VALID',
                                        dimension_numbers=('NCHW', 'OIHW', 'NCHW'))


def _matmul_bias_kernel(x_ref, w_ref, b_ref, o_ref):
    o_ref[...] = jnp.dot(x_ref[...], w_ref[...],
                         preferred_element_type=jnp.float32) + b_ref[...]


def _pallas_matmul_bias(x, w, b, block_rows=2048):
    n, cin = x.shape
    cout = w.shape[1]
    grid = (n // block_rows,)
    return pl.pallas_call(
        _matmul_bias_kernel,
        grid=grid,
        in_specs=[
            pl.BlockSpec((block_rows, cin), lambda i: (i, 0)),
            pl.BlockSpec((cin, cout), lambda i: (0, 0)),
            pl.BlockSpec((cout,), lambda i: (0,)),
        ],
        out_specs=pl.BlockSpec((block_rows, cout), lambda i: (i, 0)),
        out_shape=jax.ShapeDtypeStruct((n, cout), jnp.float32),
    )(x, w, b)


_NPIX = _HRV * _WRV
_QBLK = 1024


def _gelu_erf(x):
    return 0.5 * x * (1.0 + lax.erf(x * (1.0 / math.sqrt(2.0))))


def _query_prep_kernel(x_ref, dn_ref, rx_ref, ry_ref, pqwT_ref, pqb_ref,
                       w1qT_ref, w1d_ref, b1_ref, w2T_ref, b2_ref,
                       soxT_ref, sobx_ref, soyT_ref, soby_ref,
                       awT_ref, awb_ref, q_ref, px_ref, py_ref, awl_ref):
    q0 = jnp.dot(x_ref[...], pqwT_ref[...],
                 preferred_element_type=jnp.float32) + pqb_ref[...]
    for k in range(_K):
        dnk = dn_ref[k]
        h1 = _gelu_erf(jnp.dot(q0, w1qT_ref[...], preferred_element_type=jnp.float32)
                   + dnk * w1d_ref[...] + b1_ref[...])
        qk = jnp.dot(h1, w2T_ref[...], preferred_element_type=jnp.float32) + b2_ref[...]
        q_ref[k] = qk
        px_ref[k] = (rx_ref[k] * _WBEV - 0.5
                     + jnp.dot(qk, soxT_ref[...], preferred_element_type=jnp.float32)
                     + sobx_ref[...])
        py_ref[k] = (ry_ref[k] * _HBEV - 0.5
                     + jnp.dot(qk, soyT_ref[...], preferred_element_type=jnp.float32)
                     + soby_ref[...])
        awl_ref[k] = jnp.dot(qk, awT_ref[...],
                             preferred_element_type=jnp.float32) + awb_ref[...]


def _query_prep(x_pix, dn, rx, ry, pq_w, pq_b, qd_w1, qd_b1, qd_w2, qd_b2,
                so_w, so_b, aw_w, aw_b):
    nso = _NHEADS * _NPOINTS
    w1qT = qd_w1[:, :_D].T
    w1d = qd_w1[:, _D]
    soxT = so_w[0::2].T
    sobx = so_b[0::2]
    soyT = so_w[1::2].T
    soby = so_b[1::2]
    grid = (_NPIX // _QBLK,)
    full = lambda shape: pl.BlockSpec(shape, lambda i: tuple(0 for _ in shape))
    blk3 = lambda c: pl.BlockSpec((_K, _QBLK, c), lambda i: (0, i, 0))
    return pl.pallas_call(
        _query_prep_kernel,
        grid=grid,
        in_specs=[
            pl.BlockSpec((_QBLK, _CRV), lambda i: (i, 0)),
            blk3(1), blk3(1), blk3(1),
            full((_CRV, _D)), full((_D,)),
            full((_D, _D)), full((_D,)), full((_D,)),
            full((_D, _D)), full((_D,)),
            full((_D, nso)), full((nso,)),
            full((_D, nso)), full((nso,)),
            full((_D, nso)), full((nso,)),
        ],
        out_specs=[blk3(_D), blk3(nso), blk3(nso), blk3(nso)],
        out_shape=[
            jax.ShapeDtypeStruct((_K, _NPIX, _D), jnp.float32),
            jax.ShapeDtypeStruct((_K, _NPIX, nso), jnp.float32),
            jax.ShapeDtypeStruct((_K, _NPIX, nso), jnp.float32),
            jax.ShapeDtypeStruct((_K, _NPIX, nso), jnp.float32),
        ],
    )(x_pix, dn, rx, ry, pq_w.T, pq_b, w1qT, w1d, qd_b1, qd_w2.T, qd_b2,
      soxT, sobx, soyT, soby, aw_w.T, aw_b)


def _combine_kernel(samp_ref, q_ref, wgt_ref, opwT_ref, opb_ref,
                    powT_ref, pob_ref, o_ref):
    y = jnp.zeros((_QBLK, _D), jnp.float32)
    for k in range(_K):
        m = (jnp.dot(samp_ref[k], opwT_ref[...],
                     preferred_element_type=jnp.float32)
             + opb_ref[...] + q_ref[k])
        y = y + wgt_ref[k] * m
    o_ref[...] = jnp.dot(y, powT_ref[...],
                         preferred_element_type=jnp.float32) + pob_ref[...]


def _combine(samp, query, wgt, op_w, op_b, po_w, po_b):
    grid = (_NPIX // _QBLK,)
    full = lambda shape: pl.BlockSpec(shape, lambda i: tuple(0 for _ in shape))
    blk3 = lambda c: pl.BlockSpec((_K, _QBLK, c), lambda i: (0, i, 0))
    return pl.pallas_call(
        _combine_kernel,
        grid=grid,
        in_specs=[
            blk3(_D), blk3(_D), blk3(1),
            full((_D, _D)), full((_D,)),
            full((_D, _COUT)), full((_COUT,)),
        ],
        out_specs=pl.BlockSpec((_QBLK, _COUT), lambda i: (i, 0)),
        out_shape=jax.ShapeDtypeStruct((_NPIX, _COUT), jnp.float32),
    )(samp, query, wgt, op_w.T, op_b, po_w.T, po_b)


def kernel(x_rv, bev, lidar2ego_mat, pq_w, pq_b, pv_w, pv_b, po_w, po_b,
           qd_w1, qd_b1, qd_w2, qd_b2, rh_w1, rh_b1, rh_g1, rh_be1, rh_w2,
           rh_g2, rh_be2, rh_w3, rh_b3, so_w, so_b, aw_w, aw_b, vp_w, vp_b,
           op_w, op_b):
    b = x_rv.shape[0]
    x = jnp.transpose(x_rv, (0, 3, 1, 2))
    range_in = jnp.concatenate([x, jnp.broadcast_to(_U_VEC, (b, 3, _HRV, _WRV))], axis=1)
    h = _gelu(_gn(_conv1x1(range_in, rh_w1, rh_b1), rh_g1, rh_be1, 8))
    h = _gelu(_gn(_conv3x3_circ(h, rh_w2), rh_g2, rh_be2, 8))
    depth_logits = _conv1x1(h, rh_w3, rh_b3)
    depth_dist = jax.nn.softmax(depth_logits, axis=1)
    tp, ti = jax.lax.top_k(jnp.transpose(depth_dist, (0, 2, 3, 1)), _K)
    topk_prob = jnp.transpose(tp, (0, 3, 1, 2))
    topk_idx = jnp.transpose(ti, (0, 3, 1, 2))
    topk_depths = jnp.minimum(_BIN * (topk_idx.astype(jnp.float32) + 0.5), _RMAX - 0.5 * _BIN)
    x_l = topk_depths * _U_X
    y_l = topk_depths * _U_Y
    z_l = topk_depths * _U_Z
    p = jnp.stack([x_l, y_l, z_l, jnp.ones_like(x_l)], axis=-1).reshape(b, _NQ, 4)
    p_ego = jnp.matmul(p, lidar2ego_mat)
    rx = (p_ego[..., 0] - _XMIN) / (_XMAX - _XMIN)
    ry = (p_ego[..., 1] - _YMIN) / (_YMAX - _YMIN)
    ref_x = jnp.clip(rx, 0.0, 1.0).reshape(_K, _NPIX, 1)
    ref_y = jnp.clip(ry, 0.0, 1.0).reshape(_K, _NPIX, 1)
    dn = (topk_depths / _RMAX).reshape(_K, _NPIX, 1)

    x_pix = x_rv.reshape(_NPIX, _CRV)
    query, px, py, awl = _query_prep(
        x_pix, dn, ref_x, ref_y, pq_w, pq_b, qd_w1, qd_b1, qd_w2, qd_b2,
        so_w, so_b, aw_w, aw_b)

    wv = pv_w.T @ vp_w.T
    bv = pv_b @ vp_w.T + vp_b
    bev_flat = jnp.transpose(bev.reshape(_CBEV, _HBEV * _WBEV), (1, 0))
    v_tab = _pallas_matmul_bias(bev_flat, wv, bv, block_rows=2048)
    table = jnp.transpose(v_tab.reshape(_HBEV * _WBEV, _NHEADS, _DH),
                          (1, 0, 2)).reshape(_NHEADS * _HBEV * _WBEV, _DH)

    aw = jax.nn.softmax(awl.reshape(_NQ, _NHEADS, _NPOINTS), axis=-1)
    samp = _msda_sample_sc(table, px.reshape(_T), py.reshape(_T),
                           aw.reshape(_T))

    wgt = topk_prob / (topk_prob.sum(axis=1, keepdims=True) + 1e-8)
    y_out = _combine(samp.reshape(_K, _NPIX, _D), query,
                     wgt.reshape(_K, _NPIX, 1), op_w, op_b, po_w, po_b)
    return y_out.reshape(b, _HRV, _WRV, _COUT), depth_logits


# speedup vs baseline: 1.3171x; 1.3171x over previous
import math
import functools

import jax
import jax.numpy as jnp
import numpy as np
from jax import lax
from jax.experimental import pallas as pl
from jax.experimental.pallas import tpu as pltpu
from jax.experimental.pallas import tpu_sc as plsc

_ELEV = np.array([-30.67, -29.33, -28.0, -26.66, -25.33, -24.0, -22.67, -21.33,
                  -20.0, -18.67, -17.33, -16.0, -14.67, -13.33, -12.0, -10.67,
                  -9.33, -8.0, -6.66, -5.33, -4.0, -2.67, -1.33, 0.0, 1.33,
                  2.67, 4.0, 5.33, 6.67, 8.0, 9.33, 10.67], dtype=np.float32)
_B, _HRV, _WRV, _CRV = 1, 32, 512, 64
_CBEV, _HBEV, _WBEV = 256, 256, 256
_D, _K, _COUT = 128, 5, 128
_RMAX, _BIN, _NBINS = 55.0, 0.5, 110
_XMIN, _XMAX, _YMIN, _YMAX = -55.0, 55.0, -55.0, 55.0
_NHEADS, _NPOINTS = 8, 6
_DH = _D // _NHEADS
_NQ = _K * _HRV * _WRV

_az_line = np.linspace(-math.pi, math.pi, _WRV + 1).astype(np.float32)[:-1]
_az = np.broadcast_to(_az_line.reshape(1, 1, 1, _WRV), (1, 1, _HRV, _WRV)) \
    + 0.5 * (_az_line[1] - _az_line[0])
_elev = np.deg2rad(np.array(np.split(_ELEV[::-1].copy(), _HRV)).mean(axis=1)
                   .repeat(_WRV).reshape(_HRV, _WRV)).astype(np.float32)
_U_X = (np.cos(_az) * np.cos(_elev)).astype(np.float32)
_U_Y = (np.sin(_az) * np.cos(_elev)).astype(np.float32)
_U_Z = np.sin(_elev)[None, None].astype(np.float32)
_U_VEC = np.concatenate([_U_X, _U_Y, _U_Z], axis=1)

_NC, _NS = 2, 16
_NW = _NC * _NS
_T = _NQ * _NHEADS * _NPOINTS
_TW = _T // _NW
_STEP_IT = 96
_SS_STEPS = 16
_SS_IT = _STEP_IT * _SS_STEPS
_N_SS = _TW // _SS_IT
_GROUPS = _T // _NPOINTS
_GW = _GROUPS // _NW
_SS_G = _SS_IT // _NPOINTS


def _msda_sample_sc(table, px, py, aw):
    mesh = plsc.VectorSubcoreMesh(core_axis_name="c", subcore_axis_name="s")

    @functools.partial(
        pl.kernel, mesh=mesh,
        out_type=jax.ShapeDtypeStruct((_GROUPS, _DH), jnp.float32),
        compiler_params=pltpu.CompilerParams(needs_layout_passes=False,
                                             use_tc_tiling_on_sc=False),
        scratch_types=[
            pltpu.VMEM((_SS_IT,), jnp.float32),
            pltpu.VMEM((_SS_IT,), jnp.float32),
            pltpu.VMEM((_SS_IT,), jnp.float32),
            pltpu.VMEM((3, 128), jnp.int32),
            pltpu.VMEM((4 * _STEP_IT,), jnp.float32),
            pltpu.VMEM((4 * _STEP_IT, _DH), jnp.float32),
            pltpu.VMEM((_SS_G, _DH), jnp.float32),
            pltpu.SemaphoreType.DMA,
        ],
    )
    def k(table_h, px_h, py_h, aw_h, out_h,
          pxb, pyb, awb, idxb, sb, rowsb, outb, sem):
        wid = lax.axis_index("s") * _NC + lax.axis_index("c")
        t0w = wid * _TW
        g0w = wid * _GW
        iota = lax.iota(jnp.int32, 16)
        dcols = [jnp.bitwise_and(iota + k, 15) for k in range(_DH)]

        def ss_body(ss, carry):
            t0 = t0w + ss * _SS_IT
            pltpu.sync_copy(px_h.at[pl.ds(t0, _SS_IT)], pxb)
            pltpu.sync_copy(py_h.at[pl.ds(t0, _SS_IT)], pyb)
            pltpu.sync_copy(aw_h.at[pl.ds(t0, _SS_IT)], awb)

            def phase1(st):
                for i in range(_STEP_IT // 16):
                    o = st * _STEP_IT + i * 16
                    ivec = o + iota
                    pxv = plsc.load_gather(pxb, [ivec])
                    pyv = plsc.load_gather(pyb, [ivec])
                    awv = plsc.load_gather(awb, [ivec])
                    t_v = t0 + ivec
                    h_v = jnp.bitwise_and(t_v // _NPOINTS, _NHEADS - 1)
                    hbv = jnp.left_shift(h_v, 16)
                    xt = pxv.astype(jnp.int32)
                    xtf = xt.astype(jnp.float32)
                    x0i = xt - jnp.where(pxv < xtf, 1, 0)
                    wx1 = pxv - x0i.astype(jnp.float32)
                    wx0 = 1.0 - wx1
                    yt = pyv.astype(jnp.int32)
                    ytf = yt.astype(jnp.float32)
                    y0i = yt - jnp.where(pyv < ytf, 1, 0)
                    wy1 = pyv - y0i.astype(jnp.float32)
                    wy0 = 1.0 - wy1
                    for c, (dx, dy) in enumerate(((0, 0), (1, 0), (0, 1), (1, 1))):
                        xi = x0i + dx
                        yi = y0i + dy
                        wx = wx1 if dx else wx0
                        wy = wy1 if dy else wy0
                        valid = ((xi >= 0) & (xi <= _WBEV - 1)
                                 & (yi >= 0) & (yi <= _HBEV - 1))
                        xc = jnp.clip(xi, 0, _WBEV - 1)
                        yc = jnp.clip(yi, 0, _HBEV - 1)
                        idx_c = hbv + yc * _WBEV + xc
                        s_c = jnp.where(valid, wx * wy * awv, 0.0)
                        flat = i * 64 + c * 16
                        idxb[flat // 128, pl.ds(flat % 128, 16)] = idx_c
                        sb[pl.ds(flat, 16)] = s_c

            def fire():
                return [pltpu.async_copy(table_h.at[idxb.at[c]],
                                         rowsb.at[pl.ds(c * 128, 128)], sem)
                        for c in range(3)]

            rbases = []
            for j6 in range(_NPOINTS):
                item_v = iota * _NPOINTS + j6
                i_v = jnp.right_shift(item_v, 4)
                lane_v = jnp.bitwise_and(item_v, 15)
                rbases.append(i_v * 64 + lane_v)

            def phase3(st):
                accs = [jnp.zeros((16,), jnp.float32) for _ in range(_DH)]
                for j in range(24):
                    j6, c = j // 4, j % 4
                    r_v = rbases[j6] + c * 16
                    s_j = plsc.load_gather(sb, [r_v])
                    for k in range(_DH):
                        g_v = plsc.load_gather(rowsb, [r_v, dcols[k]])
                        accs[k] = accs[k] + s_j * g_v
                grow = st * 16 + iota
                for k in range(_DH):
                    plsc.store_scatter(outb, [grow, dcols[k]], accs[k])

            def step_body(st, c2):
                phase1(st)
                for cp in fire():
                    cp.wait()
                phase3(st)
                return c2

            lax.fori_loop(0, _SS_STEPS, step_body, 0)
            pltpu.sync_copy(outb, out_h.at[pl.ds(g0w + ss * _SS_G, _SS_G)])
            return carry

        lax.fori_loop(0, _N_SS, ss_body, 0)

    return k(table, px, py, aw)


def _conv1x1(x, w, b=None):
    y = jnp.einsum('bchw,oc->bohw', x, w)
    if b is not None:
        y = y + b[None, :, None, None]
    return y


def _gn(x, gamma, beta, groups, eps=1e-5):
    n, c, h, w = x.shape
    xg = x.reshape(n, groups, c // groups, h, w)
    m = xg.mean(axis=(2, 3, 4), keepdims=True)
    v = ((xg - m) ** 2).mean(axis=(2, 3, 4), keepdims=True)
    xg = (xg - m) / jnp.sqrt(v + eps)
    return xg.reshape(n, c, h, w) * gamma[None, :, None, None] + beta[None, :, None, None]


def _gelu(x):
    return jax.nn.gelu(x, approximate=False)


def _conv3x3_circ(x, w):
    xp = jnp.pad(x, ((0, 0), (0, 0), (1, 1), (1, 1)), mode='wrap')
    return jax.lax.conv_general_dilated(xp, w, (1, 1), 'VALID',
                                        dimension_numbers=('NCHW', 'OIHW', 'NCHW'))


def _matmul_bias_kernel(x_ref, w_ref, b_ref, o_ref):
    o_ref[...] = jnp.dot(x_ref[...], w_ref[...],
                         preferred_element_type=jnp.float32) + b_ref[...]


def _pallas_matmul_bias(x, w, b, block_rows=2048):
    n, cin = x.shape
    cout = w.shape[1]
    grid = (n // block_rows,)
    return pl.pallas_call(
        _matmul_bias_kernel,
        grid=grid,
        in_specs=[
            pl.BlockSpec((block_rows, cin), lambda i: (i, 0)),
            pl.BlockSpec((cin, cout), lambda i: (0, 0)),
            pl.BlockSpec((cout,), lambda i: (0,)),
        ],
        out_specs=pl.BlockSpec((block_rows, cout), lambda i: (i, 0)),
        out_shape=jax.ShapeDtypeStruct((n, cout), jnp.float32),
    )(x, w, b)


_NPIX = _HRV * _WRV
_QBLK = 1024


def _gelu_erf(x):
    return 0.5 * x * (1.0 + lax.erf(x * (1.0 / math.sqrt(2.0))))


def _query_prep_kernel(x_ref, dn_ref, rx_ref, ry_ref, pqwT_ref, pqb_ref,
                       w1qT_ref, w1d_ref, b1_ref, w2T_ref, b2_ref,
                       soxT_ref, sobx_ref, soyT_ref, soby_ref,
                       awT_ref, awb_ref, q_ref, px_ref, py_ref, awl_ref):
    q0 = jnp.dot(x_ref[...], pqwT_ref[...],
                 preferred_element_type=jnp.float32) + pqb_ref[...]
    for k in range(_K):
        dnk = dn_ref[k]
        h1 = _gelu_erf(jnp.dot(q0, w1qT_ref[...], preferred_element_type=jnp.float32)
                   + dnk * w1d_ref[...] + b1_ref[...])
        qk = jnp.dot(h1, w2T_ref[...], preferred_element_type=jnp.float32) + b2_ref[...]
        q_ref[k] = qk
        px_ref[k] = (rx_ref[k] * _WBEV - 0.5
                     + jnp.dot(qk, soxT_ref[...], preferred_element_type=jnp.float32)
                     + sobx_ref[...])
        py_ref[k] = (ry_ref[k] * _HBEV - 0.5
                     + jnp.dot(qk, soyT_ref[...], preferred_element_type=jnp.float32)
                     + soby_ref[...])
        awl_ref[k] = jnp.dot(qk, awT_ref[...],
                             preferred_element_type=jnp.float32) + awb_ref[...]


def _query_prep(x_pix, dn, rx, ry, pq_w, pq_b, qd_w1, qd_b1, qd_w2, qd_b2,
                so_w, so_b, aw_w, aw_b):
    nso = _NHEADS * _NPOINTS
    w1qT = qd_w1[:, :_D].T
    w1d = qd_w1[:, _D]
    soxT = so_w[0::2].T
    sobx = so_b[0::2]
    soyT = so_w[1::2].T
    soby = so_b[1::2]
    grid = (_NPIX // _QBLK,)
    full = lambda shape: pl.BlockSpec(shape, lambda i: tuple(0 for _ in shape))
    blk3 = lambda c: pl.BlockSpec((_K, _QBLK, c), lambda i: (0, i, 0))
    return pl.pallas_call(
        _query_prep_kernel,
        grid=grid,
        in_specs=[
            pl.BlockSpec((_QBLK, _CRV), lambda i: (i, 0)),
            blk3(1), blk3(1), blk3(1),
            full((_CRV, _D)), full((_D,)),
            full((_D, _D)), full((_D,)), full((_D,)),
            full((_D, _D)), full((_D,)),
            full((_D, nso)), full((nso,)),
            full((_D, nso)), full((nso,)),
            full((_D, nso)), full((nso,)),
        ],
        out_specs=[blk3(_D), blk3(nso), blk3(nso), blk3(nso)],
        out_shape=[
            jax.ShapeDtypeStruct((_K, _NPIX, _D), jnp.float32),
            jax.ShapeDtypeStruct((_K, _NPIX, nso), jnp.float32),
            jax.ShapeDtypeStruct((_K, _NPIX, nso), jnp.float32),
            jax.ShapeDtypeStruct((_K, _NPIX, nso), jnp.float32),
        ],
    )(x_pix, dn, rx, ry, pq_w.T, pq_b, w1qT, w1d, qd_b1, qd_w2.T, qd_b2,
      soxT, sobx, soyT, soby, aw_w.T, aw_b)


def _combine_kernel(samp_ref, q_ref, wgt_ref, opwT_ref, opb_ref,
                    powT_ref, pob_ref, o_ref):
    y = jnp.zeros((_QBLK, _D), jnp.float32)
    for k in range(_K):
        m = (jnp.dot(samp_ref[k], opwT_ref[...],
                     preferred_element_type=jnp.float32)
             + opb_ref[...] + q_ref[k])
        y = y + wgt_ref[k] * m
    o_ref[...] = jnp.dot(y, powT_ref[...],
                         preferred_element_type=jnp.float32) + pob_ref[...]


def _combine(samp, query, wgt, op_w, op_b, po_w, po_b):
    grid = (_NPIX // _QBLK,)
    full = lambda shape: pl.BlockSpec(shape, lambda i: tuple(0 for _ in shape))
    blk3 = lambda c: pl.BlockSpec((_K, _QBLK, c), lambda i: (0, i, 0))
    return pl.pallas_call(
        _combine_kernel,
        grid=grid,
        in_specs=[
            blk3(_D), blk3(_D), blk3(1),
            full((_D, _D)), full((_D,)),
            full((_D, _COUT)), full((_COUT,)),
        ],
        out_specs=pl.BlockSpec((_QBLK, _COUT), lambda i: (i, 0)),
        out_shape=jax.ShapeDtypeStruct((_NPIX, _COUT), jnp.float32),
    )(samp, query, wgt, op_w.T, op_b, po_w.T, po_b)


def kernel(x_rv, bev, lidar2ego_mat, pq_w, pq_b, pv_w, pv_b, po_w, po_b,
           qd_w1, qd_b1, qd_w2, qd_b2, rh_w1, rh_b1, rh_g1, rh_be1, rh_w2,
           rh_g2, rh_be2, rh_w3, rh_b3, so_w, so_b, aw_w, aw_b, vp_w, vp_b,
           op_w, op_b):
    b = x_rv.shape[0]
    x = jnp.transpose(x_rv, (0, 3, 1, 2))
    range_in = jnp.concatenate([x, jnp.broadcast_to(_U_VEC, (b, 3, _HRV, _WRV))], axis=1)
    h = _gelu(_gn(_conv1x1(range_in, rh_w1, rh_b1), rh_g1, rh_be1, 8))
    h = _gelu(_gn(_conv3x3_circ(h, rh_w2), rh_g2, rh_be2, 8))
    depth_logits = _conv1x1(h, rh_w3, rh_b3)
    depth_dist = jax.nn.softmax(depth_logits, axis=1)
    tp, ti = jax.lax.top_k(jnp.transpose(depth_dist, (0, 2, 3, 1)), _K)
    topk_prob = jnp.transpose(tp, (0, 3, 1, 2))
    topk_idx = jnp.transpose(ti, (0, 3, 1, 2))
    topk_depths = jnp.minimum(_BIN * (topk_idx.astype(jnp.float32) + 0.5), _RMAX - 0.5 * _BIN)
    x_l = topk_depths * _U_X
    y_l = topk_depths * _U_Y
    z_l = topk_depths * _U_Z
    p = jnp.stack([x_l, y_l, z_l, jnp.ones_like(x_l)], axis=-1).reshape(b, _NQ, 4)
    p_ego = jnp.matmul(p, lidar2ego_mat)
    rx = (p_ego[..., 0] - _XMIN) / (_XMAX - _XMIN)
    ry = (p_ego[..., 1] - _YMIN) / (_YMAX - _YMIN)
    ref_x = jnp.clip(rx, 0.0, 1.0).reshape(_K, _NPIX, 1)
    ref_y = jnp.clip(ry, 0.0, 1.0).reshape(_K, _NPIX, 1)
    dn = (topk_depths / _RMAX).reshape(_K, _NPIX, 1)

    x_pix = x_rv.reshape(_NPIX, _CRV)
    query, px, py, awl = _query_prep(
        x_pix, dn, ref_x, ref_y, pq_w, pq_b, qd_w1, qd_b1, qd_w2, qd_b2,
        so_w, so_b, aw_w, aw_b)

    wv = pv_w.T @ vp_w.T
    bv = pv_b @ vp_w.T + vp_b
    bev_flat = jnp.transpose(bev.reshape(_CBEV, _HBEV * _WBEV), (1, 0))
    v_tab = _pallas_matmul_bias(bev_flat, wv, bv, block_rows=2048)
    table = jnp.transpose(v_tab.reshape(_HBEV * _WBEV, _NHEADS, _DH),
                          (1, 0, 2)).reshape(_NHEADS * _HBEV * _WBEV, _DH)

    aw = jax.nn.softmax(awl.reshape(_NQ, _NHEADS, _NPOINTS), axis=-1)
    samp = _msda_sample_sc(table, px.reshape(_T), py.reshape(_T),
                           aw.reshape(_T))

    wgt = topk_prob / (topk_prob.sum(axis=1, keepdims=True) + 1e-8)
    y_out = _combine(samp.reshape(_K, _NPIX, _D), query,
                     wgt.reshape(_K, _NPIX, 1), op_w, op_b, po_w, po_b)
    return y_out.reshape(b, _HRV, _WRV, _COUT), depth_logits
